# baseline (speedup 1.0000x reference)
import jax
import jax.numpy as jnp
from jax.experimental import pallas as pl
from jax.experimental.pallas import tpu as pltpu

C_IN = 64
C_OUT = 64
TILE = 512


def _mlp_matmul_kernel(locs_ref, w0_ref, w1_ref, w2_ref, x_ref, y_ref):
    h = jnp.sin(jnp.dot(locs_ref[...], w0_ref[...], preferred_element_type=jnp.float32))
    h = jnp.sin(jnp.dot(h, w1_ref[...], preferred_element_type=jnp.float32))
    g = jnp.dot(h, w2_ref[...], preferred_element_type=jnp.float32)
    g3 = g.reshape(TILE, C_IN, C_OUT)
    x = x_ref[...]
    y = jax.lax.dot_general(
        x, g3,
        dimension_numbers=(((2,), (1,)), ((1,), (0,))),
        preferred_element_type=jnp.float32,
    )
    y_ref[...] = y.transpose(1, 0, 2)


def kernel(features, eval_locs, W0, W1, W2, eval_indices):
    b, c_in, n_in = features.shape
    nnz = eval_indices.shape[0]
    np_pad = ((nnz + TILE - 1) // TILE) * TILE

    idx0 = eval_indices[:, 0].astype(jnp.int32)
    idx1 = eval_indices[:, 1].astype(jnp.int32)

    gathered = features[:, :, idx1].reshape(b, c_in * nnz)
    xf = jnp.pad(gathered, ((0, 0), (0, np_pad * C_IN - c_in * nnz)))
    x3 = xf.reshape(b, np_pad, C_IN)

    locs_pad = jnp.pad(eval_locs, ((0, np_pad - nnz), (0, 0)))

    y3 = pl.pallas_call(
        _mlp_matmul_kernel,
        grid=(np_pad // TILE,),
        in_specs=[
            pl.BlockSpec((TILE, 2), lambda i: (i, 0)),
            pl.BlockSpec((2, 64), lambda i: (0, 0)),
            pl.BlockSpec((64, 64), lambda i: (0, 0)),
            pl.BlockSpec((64, C_IN * C_OUT), lambda i: (0, 0)),
            pl.BlockSpec((b, TILE, C_IN), lambda i: (0, i, 0)),
        ],
        out_specs=pl.BlockSpec((b, TILE, C_OUT), lambda i: (0, i, 0)),
        out_shape=jax.ShapeDtypeStruct((b, np_pad, C_OUT), jnp.float32),
    )(locs_pad, W0, W1, W2, x3)

    vf = y3.reshape(b, np_pad * C_OUT)[:, : c_in * nnz].reshape(b, C_OUT, nnz)
    n_out = 1024
    integral = jnp.zeros((b, C_OUT, n_out), dtype=features.dtype).at[:, :, idx0].add(vf)
    return integral


# speedup vs baseline: 7.0675x; 7.0675x over previous
import dataclasses
import functools

import jax
import jax.numpy as jnp
from jax import lax
from jax.experimental import pallas as pl
from jax.experimental.pallas import tpu as pltpu
from jax.experimental.pallas import tpu_sc as plsc

C_IN = 64
C_OUT = 64
TILE = 512
L = 16


def _sc_compiler_params():
    cp = pltpu.CompilerParams()
    if "needs_layout_passes" in pltpu.CompilerParams.__dataclass_fields__:
        cp = dataclasses.replace(cp, needs_layout_passes=False)
    return cp


def _mlp_matmul_kernel(locs_ref, w0_ref, w1_ref, w2_ref, x_ref, y_ref):
    h = jnp.sin(jnp.dot(locs_ref[...], w0_ref[...], preferred_element_type=jnp.float32))
    h = jnp.sin(jnp.dot(h, w1_ref[...], preferred_element_type=jnp.float32))
    g = jnp.dot(h, w2_ref[...], preferred_element_type=jnp.float32)
    g3 = g.reshape(TILE, C_IN, C_OUT)
    x = x_ref[...]
    y = jax.lax.dot_general(
        x, g3,
        dimension_numbers=(((2,), (1,)), ((1,), (0,))),
        preferred_element_type=jnp.float32,
    )
    y_ref[...] = y.transpose(1, 0, 2)


def _sc_gather(feat_flat, idx1p, b, n_in, nnz, np64):
    nv = idx1p.shape[0]
    nch = nv // L
    mesh = plsc.VectorSubcoreMesh(core_axis_name="c", subcore_axis_name="s")

    @functools.partial(
        pl.kernel,
        out_type=jax.ShapeDtypeStruct((b * np64,), jnp.float32),
        mesh=mesh,
        scratch_types=[
            pltpu.VMEM((nv,), jnp.int32),
            pltpu.VMEM((2 * n_in,), jnp.float32),
            pltpu.VMEM((2 * nv,), jnp.float32),
            pltpu.SemaphoreType.DMA,
        ],
        compiler_params=_sc_compiler_params(),
    )
    def gather_kernel(feat_hbm, idx_hbm, x_hbm, idx_v, src_v, dst_v, sem):
        w = lax.axis_index("s") * 2 + lax.axis_index("c")
        pltpu.sync_copy(idx_hbm, idx_v)
        iota16 = lax.iota(jnp.int32, L)

        @pl.loop(0, b)
        def _batch(bi):
            pltpu.async_copy(
                feat_hbm.at[pl.ds((bi * C_IN + 2 * w) * n_in, 2 * n_in)],
                src_v, sem,
            ).wait()

            @pl.loop(0, nch)
            def _ch0(i):
                idx = idx_v[pl.ds(i * L, L)]
                v = plsc.load_gather(src_v, [idx])
                dst_v[pl.ds(i * L, L)] = v

            @pl.loop(0, nch)
            def _ch1(i):
                idx = idx_v[pl.ds(i * L, L)] + n_in
                v = plsc.load_gather(src_v, [idx])
                plsc.store_scatter(dst_v, [iota16 + (nnz + i * L)], v)

            pltpu.async_copy(
                dst_v.at[pl.ds(0, 2 * nnz)],
                x_hbm.at[pl.ds(bi * np64 + w * 2 * nnz, 2 * nnz)],
                sem,
            ).wait()

    return gather_kernel(feat_flat, idx1p)


def _sc_scatter(y_flat, idx0p, b, nnz, np64, n_out):
    nv = idx0p.shape[0]
    nch = nv // L
    acc_w = n_out + L
    mesh = plsc.VectorSubcoreMesh(core_axis_name="c", subcore_axis_name="s")

    @functools.partial(
        pl.kernel,
        out_type=jax.ShapeDtypeStruct((b * C_OUT * n_out,), jnp.float32),
        mesh=mesh,
        scratch_types=[
            pltpu.VMEM((nv,), jnp.int32),
            pltpu.VMEM((2 * nv,), jnp.float32),
            pltpu.VMEM((L * acc_w,), jnp.float32),
            pltpu.VMEM((n_out,), jnp.float32),
            pltpu.SemaphoreType.DMA,
        ],
        compiler_params=_sc_compiler_params(),
    )
    def scatter_kernel(y_hbm, idx_hbm, out_hbm, idx_v, val_v, acc, obuf, sem):
        w = lax.axis_index("s") * 2 + lax.axis_index("c")
        pltpu.sync_copy(idx_hbm, idx_v)
        zeros16f = jnp.zeros((L,), jnp.float32)
        iota16 = lax.iota(jnp.int32, L)
        lanebase = iota16 * acc_w

        @pl.loop(0, b)
        def _batch(bi):
            pltpu.async_copy(
                y_hbm.at[pl.ds(bi * np64 + w * 2 * nnz, 2 * nnz)],
                val_v.at[pl.ds(0, 2 * nnz)],
                sem,
            ).wait()

            for ch in range(2):
                @pl.loop(0, L * acc_w // L)
                def _zero(z):
                    acc[pl.ds(z * L, L)] = zeros16f

                if ch == 0:
                    @pl.loop(0, nch)
                    def _acc0(i):
                        p = idx_v[pl.ds(i * L, L)]
                        v = val_v[pl.ds(i * L, L)]
                        plsc.addupdate_scatter(acc, [lanebase + p], v)
                else:
                    @pl.loop(0, nch)
                    def _acc1(i):
                        p = idx_v[pl.ds(i * L, L)]
                        v = plsc.load_gather(val_v, [iota16 + (nnz + i * L)])
                        plsc.addupdate_scatter(acc, [lanebase + p], v)

                @pl.loop(0, n_out // L)
                def _reduce(j):
                    s = acc[pl.ds(j * L, L)]
                    for lane in range(1, L):
                        s = s + acc[pl.ds(lane * acc_w + j * L, L)]
                    obuf[pl.ds(j * L, L)] = s

                pltpu.sync_copy(
                    obuf,
                    out_hbm.at[pl.ds((bi * C_OUT + 2 * w + ch) * n_out, n_out)],
                )

    return scatter_kernel(y_flat, idx0p)


def kernel(features, eval_locs, W0, W1, W2, eval_indices):
    b, c_in, n_in = features.shape
    nnz = eval_indices.shape[0]
    np_pad = ((nnz + TILE - 1) // TILE) * TILE
    nv = ((nnz + L - 1) // L) * L
    n_out = 1024

    idx0 = eval_indices[:, 0].astype(jnp.int32)
    idx1 = eval_indices[:, 1].astype(jnp.int32)
    idx1p = jnp.pad(idx1, (0, nv - nnz))
    idx0p = jnp.pad(idx0, (0, nv - nnz), constant_values=n_out)

    x_flat = _sc_gather(features.reshape(-1), idx1p, b, n_in, nnz, np_pad * C_IN)
    x3 = x_flat.reshape(b, np_pad, C_IN)

    locs_pad = jnp.pad(eval_locs, ((0, np_pad - nnz), (0, 0)))

    y3 = pl.pallas_call(
        _mlp_matmul_kernel,
        grid=(np_pad // TILE,),
        in_specs=[
            pl.BlockSpec((TILE, 2), lambda i: (i, 0)),
            pl.BlockSpec((2, 64), lambda i: (0, 0)),
            pl.BlockSpec((64, 64), lambda i: (0, 0)),
            pl.BlockSpec((64, C_IN * C_OUT), lambda i: (0, 0)),
            pl.BlockSpec((b, TILE, C_IN), lambda i: (0, i, 0)),
        ],
        out_specs=pl.BlockSpec((b, TILE, C_OUT), lambda i: (0, i, 0)),
        out_shape=jax.ShapeDtypeStruct((b, np_pad, C_OUT), jnp.float32),
    )(locs_pad, W0, W1, W2, x3)

    y_flat = y3.reshape(-1)
    out_flat = _sc_scatter(y_flat, idx0p, b, nnz, np_pad * C_OUT, n_out)
    return out_flat.reshape(b, C_OUT, n_out)


# speedup vs baseline: 7.5215x; 1.0642x over previous
import dataclasses
import functools

import jax
import jax.numpy as jnp
from jax import lax
from jax.experimental import pallas as pl
from jax.experimental.pallas import tpu as pltpu
from jax.experimental.pallas import tpu_sc as plsc

C_IN = 64
C_OUT = 64
TILE = 512
L = 16


def _sc_compiler_params():
    cp = pltpu.CompilerParams()
    if "needs_layout_passes" in pltpu.CompilerParams.__dataclass_fields__:
        cp = dataclasses.replace(cp, needs_layout_passes=False)
    return cp


def _mlp_matmul_kernel(locsT_ref, w0t_ref, w1t_ref, w2t_ref, x_ref, y_ref):
    h = jnp.sin(jnp.dot(w0t_ref[...], locsT_ref[...], preferred_element_type=jnp.float32))
    h = jnp.sin(jnp.dot(w1t_ref[...], h, preferred_element_type=jnp.float32))
    gT = jnp.dot(w2t_ref[...].astype(jnp.bfloat16), h.astype(jnp.bfloat16),
                 preferred_element_type=jnp.float32)
    gr = gT.astype(jnp.bfloat16).reshape(C_IN, C_OUT, TILE)
    x = x_ref[...].astype(jnp.bfloat16)
    y = jax.lax.dot_general(
        x, gr,
        dimension_numbers=(((2,), (0,)), ((1,), (2,))),
        preferred_element_type=jnp.float32,
    )
    y_ref[...] = y.transpose(1, 0, 2)


def _sc_gather(feat_flat, idx1p, b, n_in, nnz, np64):
    nv = idx1p.shape[0]
    nch = nv // L
    mesh = plsc.VectorSubcoreMesh(core_axis_name="c", subcore_axis_name="s")

    @functools.partial(
        pl.kernel,
        out_type=jax.ShapeDtypeStruct((b * np64,), jnp.float32),
        mesh=mesh,
        scratch_types=[
            pltpu.VMEM((nv,), jnp.int32),
            pltpu.VMEM((2 * n_in,), jnp.float32),
            pltpu.VMEM((2 * nv,), jnp.float32),
            pltpu.SemaphoreType.DMA,
        ],
        compiler_params=_sc_compiler_params(),
    )
    def gather_kernel(feat_hbm, idx_hbm, x_hbm, idx_v, src_v, dst_v, sem):
        w = lax.axis_index("s") * 2 + lax.axis_index("c")
        pltpu.sync_copy(idx_hbm, idx_v)
        iota16 = lax.iota(jnp.int32, L)

        @pl.loop(0, b)
        def _batch(bi):
            pltpu.async_copy(
                feat_hbm.at[pl.ds((bi * C_IN + 2 * w) * n_in, 2 * n_in)],
                src_v, sem,
            ).wait()

            @pl.loop(0, nch)
            def _ch0(i):
                idx = idx_v[pl.ds(i * L, L)]
                v = plsc.load_gather(src_v, [idx])
                dst_v[pl.ds(i * L, L)] = v

            @pl.loop(0, nch)
            def _ch1(i):
                idx = idx_v[pl.ds(i * L, L)] + n_in
                v = plsc.load_gather(src_v, [idx])
                plsc.store_scatter(dst_v, [iota16 + (nnz + i * L)], v)

            pltpu.async_copy(
                dst_v.at[pl.ds(0, 2 * nnz)],
                x_hbm.at[pl.ds(bi * np64 + w * 2 * nnz, 2 * nnz)],
                sem,
            ).wait()

    return gather_kernel(feat_flat, idx1p)


def _sc_scatter(y_flat, idx0p, b, nnz, np64, n_out):
    nv = idx0p.shape[0]
    nch = nv // L
    acc_w = n_out + L
    mesh = plsc.VectorSubcoreMesh(core_axis_name="c", subcore_axis_name="s")

    @functools.partial(
        pl.kernel,
        out_type=jax.ShapeDtypeStruct((b * C_OUT * n_out,), jnp.float32),
        mesh=mesh,
        scratch_types=[
            pltpu.VMEM((nv,), jnp.int32),
            pltpu.VMEM((2 * nv,), jnp.float32),
            pltpu.VMEM((L * acc_w,), jnp.float32),
            pltpu.VMEM((n_out,), jnp.float32),
            pltpu.SemaphoreType.DMA,
        ],
        compiler_params=_sc_compiler_params(),
    )
    def scatter_kernel(y_hbm, idx_hbm, out_hbm, idx_v, val_v, acc, obuf, sem):
        w = lax.axis_index("s") * 2 + lax.axis_index("c")
        pltpu.sync_copy(idx_hbm, idx_v)
        zeros16f = jnp.zeros((L,), jnp.float32)
        iota16 = lax.iota(jnp.int32, L)
        lanebase = iota16 * acc_w

        @pl.loop(0, b)
        def _batch(bi):
            pltpu.async_copy(
                y_hbm.at[pl.ds(bi * np64 + w * 2 * nnz, 2 * nnz)],
                val_v.at[pl.ds(0, 2 * nnz)],
                sem,
            ).wait()

            for ch in range(2):
                @pl.loop(0, L * acc_w // L)
                def _zero(z):
                    acc[pl.ds(z * L, L)] = zeros16f

                if ch == 0:
                    @pl.loop(0, nch)
                    def _acc0(i):
                        p = idx_v[pl.ds(i * L, L)]
                        v = val_v[pl.ds(i * L, L)]
                        plsc.addupdate_scatter(acc, [lanebase + p], v)
                else:
                    @pl.loop(0, nch)
                    def _acc1(i):
                        p = idx_v[pl.ds(i * L, L)]
                        v = plsc.load_gather(val_v, [iota16 + (nnz + i * L)])
                        plsc.addupdate_scatter(acc, [lanebase + p], v)

                @pl.loop(0, n_out // L)
                def _reduce(j):
                    s = acc[pl.ds(j * L, L)]
                    for lane in range(1, L):
                        s = s + acc[pl.ds(lane * acc_w + j * L, L)]
                    obuf[pl.ds(j * L, L)] = s

                pltpu.sync_copy(
                    obuf,
                    out_hbm.at[pl.ds((bi * C_OUT + 2 * w + ch) * n_out, n_out)],
                )

    return scatter_kernel(y_flat, idx0p)


def kernel(features, eval_locs, W0, W1, W2, eval_indices):
    b, c_in, n_in = features.shape
    nnz = eval_indices.shape[0]
    np_pad = ((nnz + TILE - 1) // TILE) * TILE
    nv = ((nnz + L - 1) // L) * L
    n_out = 1024

    idx0 = eval_indices[:, 0].astype(jnp.int32)
    idx1 = eval_indices[:, 1].astype(jnp.int32)
    idx1p = jnp.pad(idx1, (0, nv - nnz))
    idx0p = jnp.pad(idx0, (0, nv - nnz), constant_values=n_out)

    x_flat = _sc_gather(features.reshape(-1), idx1p, b, n_in, nnz, np_pad * C_IN)
    x3 = x_flat.reshape(b, np_pad, C_IN)

    locsT_pad = jnp.pad(eval_locs, ((0, np_pad - nnz), (0, 0))).T

    y3 = pl.pallas_call(
        _mlp_matmul_kernel,
        grid=(np_pad // TILE,),
        in_specs=[
            pl.BlockSpec((2, TILE), lambda i: (0, i)),
            pl.BlockSpec((64, 2), lambda i: (0, 0)),
            pl.BlockSpec((64, 64), lambda i: (0, 0)),
            pl.BlockSpec((C_IN * C_OUT, 64), lambda i: (0, 0)),
            pl.BlockSpec((b, TILE, C_IN), lambda i: (0, i, 0)),
        ],
        out_specs=pl.BlockSpec((b, TILE, C_OUT), lambda i: (0, i, 0)),
        out_shape=jax.ShapeDtypeStruct((b, np_pad, C_OUT), jnp.float32),
    )(locsT_pad, W0.T, W1.T, W2.T, x3)

    y_flat = y3.reshape(-1)
    out_flat = _sc_scatter(y_flat, idx0p, b, nnz, np_pad * C_OUT, n_out)
    return out_flat.reshape(b, C_OUT, n_out)
